# baseline (speedup 1.0000x reference)
import functools

import jax
import jax.numpy as jnp
from jax import lax
from jax.experimental import pallas as pl
from jax.experimental.pallas import tpu as pltpu
from jax.experimental.pallas import tpu_sc as plsc

SEQ = 200
EMB = 64
PADDED = 128
BATCH = 1024
NC = 2
NS = 16
NW = NC * NS
SEQ_PER_W = BATCH // NW
HALF = SEQ // 2
LANES = 16
SCALE = 8.0


def _sc_embed(idx, tbl128, pos_table):
    mesh = plsc.VectorSubcoreMesh(
        core_axis_name="c", subcore_axis_name="s", num_cores=NC, num_subcores=NS
    )

    @functools.partial(
        pl.kernel,
        mesh=mesh,
        compiler_params=pltpu.CompilerParams(use_tc_tiling_on_sc=False),
        out_type=jax.ShapeDtypeStruct((BATCH, SEQ, EMB), jnp.float32),
        scratch_types=[
            pltpu.VMEM((SEQ_PER_W, 2, HALF), jnp.int32),
            pltpu.VMEM((SEQ, EMB), jnp.float32),
            pltpu.VMEM((SEQ, PADDED), jnp.float32),
            pltpu.VMEM((SEQ, PADDED), jnp.float32),
            pltpu.VMEM((SEQ, EMB), jnp.float32),
            pltpu.VMEM((SEQ, EMB), jnp.float32),
            pltpu.SemaphoreType.DMA,
            pltpu.SemaphoreType.DMA,
            pltpu.SemaphoreType.DMA,
            pltpu.SemaphoreType.DMA,
        ],
    )
    def k(idx_hbm, tok_hbm, pos_hbm, out_hbm, idx_v, pos_v,
          g0, g1, o0, o1, gs0, gs1, ws0, ws1):
        gbufs = (g0, g1)
        obufs = (o0, o1)
        gsem = (gs0, gs1)
        wsem = (ws0, ws1)
        wid = lax.axis_index("s") * NC + lax.axis_index("c")
        base = wid * SEQ_PER_W
        pltpu.sync_copy(pos_hbm, pos_v)
        pltpu.sync_copy(idx_hbm.at[pl.ds(base, SEQ_PER_W)], idx_v)

        def start_gather(i, b):
            pltpu.async_copy(
                tok_hbm.at[idx_v.at[i, 0]], gbufs[b].at[pl.ds(0, HALF)], gsem[b]
            )
            pltpu.async_copy(
                tok_hbm.at[idx_v.at[i, 1]], gbufs[b].at[pl.ds(HALF, HALF)], gsem[b]
            )

        def wait_gather(b):
            pltpu.make_async_copy(tok_hbm.at[pl.ds(0, SEQ)], gbufs[b], gsem[b]).wait()

        def wait_wb(b):
            pltpu.make_async_copy(obufs[b], out_hbm.at[0], wsem[b]).wait()

        def compute(b):
            gbuf = gbufs[b]
            obuf = obufs[b]

            def row_body(r, carry):
                for rr in range(4):
                    row = r * 4 + rr
                    for j in range(EMB // LANES):
                        sl = pl.ds(j * LANES, LANES)
                        obuf[row, sl] = gbuf[row, sl] * SCALE + pos_v[row, sl]
                return carry

            lax.fori_loop(0, SEQ // 4, row_body, 0)

        start_gather(0, 0)

        def outer(o, carry):
            for phase in range(2):
                i = 2 * o + phase
                b = phase
                nb = 1 - phase
                if phase == 0:
                    start_gather(i + 1, nb)
                else:
                    @pl.when(o <= SEQ_PER_W // 2 - 2)
                    def _():
                        start_gather(i + 1, nb)

                wait_gather(b)

                @pl.when(o >= 1)
                def _():
                    wait_wb(b)

                compute(b)
                pltpu.async_copy(obufs[b], out_hbm.at[base + i], wsem[b])
            return carry

        lax.fori_loop(0, SEQ_PER_W // 2, outer, 0)
        wait_wb(0)
        wait_wb(1)

    return k(idx, tbl128, pos_table)


S_BLK = 8
B_BLK = 512


def _transpose_body(x_ref, o_ref):
    for s in range(S_BLK):
        o_ref[s, :, :] = x_ref[:, s, :].T


def _tc_transpose(out3):
    return pl.pallas_call(
        _transpose_body,
        grid=(SEQ // S_BLK, BATCH // B_BLK),
        in_specs=[
            pl.BlockSpec((B_BLK, S_BLK, EMB), lambda i, j: (j, i, 0)),
        ],
        out_specs=pl.BlockSpec((S_BLK, EMB, B_BLK), lambda i, j: (i, 0, j)),
        out_shape=jax.ShapeDtypeStruct((SEQ, EMB, BATCH), jnp.float32),
        compiler_params=pltpu.CompilerParams(
            dimension_semantics=("arbitrary", "arbitrary")
        ),
    )(out3)


def kernel(inputs, token_table, pos_table):
    idx = inputs.astype(jnp.int32).reshape(BATCH, 2, HALF)
    tbl128 = jnp.pad(token_table.astype(jnp.float32), ((0, 0), (0, PADDED - EMB)))
    out = _sc_embed(idx, tbl128, pos_table.astype(jnp.float32))
    out_t = _tc_transpose(out)
    return jnp.transpose(out_t, (2, 0, 1))


# speedup vs baseline: 1.5702x; 1.5702x over previous
import functools

import jax
import jax.numpy as jnp
from jax import lax
from jax.experimental import pallas as pl
from jax.experimental.pallas import tpu as pltpu
from jax.experimental.pallas import tpu_sc as plsc

SEQ = 200
EMB = 64
PADDED = 128
BATCH = 1024
NC = 2
NS = 16
NW = NC * NS
SEQ_PER_W = BATCH // NW
HALF = SEQ // 2
LANES = 16
SCALE = 8.0
NBUF = 4


def _sc_embed(idx, tblv, pos_table):
    mesh = plsc.VectorSubcoreMesh(
        core_axis_name="c", subcore_axis_name="s", num_cores=NC, num_subcores=NS
    )

    @functools.partial(
        pl.kernel,
        mesh=mesh,
        compiler_params=pltpu.CompilerParams(use_tc_tiling_on_sc=False),
        out_type=jax.ShapeDtypeStruct((BATCH, SEQ, EMB), jnp.float32),
        scratch_types=[
            pltpu.VMEM((SEQ_PER_W, 2, HALF), jnp.int32),
            pltpu.VMEM((SEQ, EMB), jnp.float32),
        ]
        + [pltpu.VMEM((SEQ, EMB), jnp.float32) for _ in range(NBUF)]
        + [pltpu.SemaphoreType.DMA for _ in range(2 * NBUF)],
    )
    def k(idx_hbm, tok_hbm, pos_hbm, out_hbm, idx_v, pos_v, *rest):
        bufs = rest[:NBUF]
        gsem = rest[NBUF : 2 * NBUF]
        wsem = rest[2 * NBUF :]
        wid = lax.axis_index("s") * NC + lax.axis_index("c")
        base = wid * SEQ_PER_W
        pltpu.sync_copy(pos_hbm, pos_v)
        pltpu.sync_copy(idx_hbm.at[pl.ds(base, SEQ_PER_W)], idx_v)

        def start_gather(i, b):
            pltpu.async_copy(
                tok_hbm.at[idx_v.at[i, 0]], bufs[b].at[pl.ds(0, HALF)], gsem[b]
            )
            pltpu.async_copy(
                tok_hbm.at[idx_v.at[i, 1]], bufs[b].at[pl.ds(HALF, HALF)], gsem[b]
            )

        def wait_gather(b):
            pltpu.make_async_copy(out_hbm.at[0], bufs[b], gsem[b]).wait()

        def wait_wb(b):
            pltpu.make_async_copy(bufs[b], out_hbm.at[0], wsem[b]).wait()

        def compute(b):
            buf = bufs[b]

            def row_body(r, carry):
                for rr in range(4):
                    row = r * 4 + rr
                    for j in range(EMB // LANES):
                        sl = pl.ds(j * LANES, LANES)
                        buf[row, sl] = buf[row, sl] * SCALE + pos_v[row, sl]
                return carry

            lax.fori_loop(0, SEQ // 4, row_body, 0)

        for i in range(NBUF - 1):
            start_gather(i, i)

        def outer(o, carry):
            for phase in range(NBUF):
                i = NBUF * o + phase
                b = phase
                nb = (phase + NBUF - 1) % NBUF
                if phase == 0:
                    @pl.when(o >= 1)
                    def _():
                        wait_wb(nb)

                    start_gather(i + NBUF - 1, nb)
                else:
                    @pl.when(o <= SEQ_PER_W // NBUF - 2)
                    def _():
                        wait_wb(nb)
                        start_gather(i + NBUF - 1, nb)

                wait_gather(b)
                compute(b)
                pltpu.async_copy(bufs[b], out_hbm.at[base + i], wsem[b])
            return carry

        lax.fori_loop(0, SEQ_PER_W // NBUF, outer, 0)
        for b in range(NBUF):
            wait_wb(b)

    return k(idx, tblv, pos_table)


def kernel(inputs, token_table, pos_table):
    idx = (inputs.astype(jnp.int32) * 2).reshape(BATCH, 2, HALF)
    tbl128 = jnp.pad(token_table.astype(jnp.float32), ((0, 0), (0, PADDED - EMB)))
    tblv = tbl128.reshape(2 * 100000, EMB)
    return _sc_embed(idx, tblv, pos_table.astype(jnp.float32))
